# baseline (speedup 1.0000x reference)
import functools

import jax
import jax.numpy as jnp
from jax import lax
from jax.experimental import pallas as pl
from jax.experimental.pallas import tpu as pltpu
from jax.experimental.pallas import tpu_sc as plsc

B, L = 4096, 200
EMB = 14
HID = 16
N = B * L
VOC = 1000000
W_CLAMP = (VOC * HID - EMB) // EMB

_info = plsc.get_sparse_core_info()
NC, NS, LANES = _info.num_cores, _info.num_subcores, _info.num_lanes
NW = NC * NS
BPW = B // NW
BB = 4
BLK = BB * L
NBLK = BPW // BB
PER_W = BPW * L
NWIN = 2 * BLK
_SEGS = [(s, min(128, NWIN - s)) for s in range(0, NWIN, 128)]

_mesh = plsc.VectorSubcoreMesh(core_axis_name="c", subcore_axis_name="s")


@functools.partial(
    pl.kernel,
    mesh=_mesh,
    out_type=jax.ShapeDtypeStruct((B, L, HID), jnp.float32),
    scratch_types=[
        pltpu.VMEM((2 * BLK,), jnp.int32),
        pltpu.VMEM((2 * BLK,), jnp.int32),
        pltpu.VMEM((2 * BLK,), jnp.int32),
        pltpu.VMEM((2 * BLK,), jnp.float32),
        pltpu.VMEM((2 * NWIN,), jnp.int32),
        pltpu.VMEM((2 * NWIN, EMB), jnp.float32),
        pltpu.VMEM((2 * BLK, HID), jnp.float32),
        pltpu.SemaphoreType.DMA,
        pltpu.SemaphoreType.DMA,
    ],
    compiler_params=pltpu.CompilerParams(
        needs_layout_passes=False,
        use_tc_tiling_on_sc=False,
    ),
)
def _sc_embed(table_h, idx_h, dur_h, out_h, idx_v, w1_v, o_v, dur_v, wl_v,
              win_v, out_v, sem_g, sem_o):
    wid = lax.axis_index("s") * NC + lax.axis_index("c")
    lane = lax.iota(jnp.int32, LANES)
    c_emb = jnp.minimum(lane, EMB - 1)
    m_emb = lane < EMB

    def stage(g, s):
        base = wid * PER_W + g * BLK
        pltpu.sync_copy(idx_h.at[pl.ds(base, BLK)], idx_v.at[pl.ds(s * BLK, BLK)])
        pltpu.sync_copy(dur_h.at[pl.ds(base, BLK)], dur_v.at[pl.ds(s * BLK, BLK)])

        @plsc.parallel_loop(0, BLK // LANES, unroll=4)
        def _(j):
            ii = idx_v[pl.ds(s * BLK + j * LANES, LANES)]
            t = (ii * 8) // 7
            w1_v[pl.ds(s * BLK + j * LANES, LANES)] = t
            o_v[pl.ds(s * BLK + j * LANES, LANES)] = ii * 16 - t * 14

        @plsc.parallel_loop(0, NWIN // LANES, unroll=4)
        def _(j):
            k = j * LANES + lane
            t = plsc.load_gather(w1_v, [s * BLK + (k >> 1)])
            wl_v[pl.ds(s * NWIN + j * LANES, LANES)] = jnp.minimum(
                t + (k & 1), W_CLAMP
            )

    def fire_gathers(s):
        for off, ln in _SEGS:
            pltpu.async_copy(
                table_h.at[wl_v.at[pl.ds(s * NWIN + off, ln)]],
                win_v.at[pl.ds(s * NWIN + off, ln)],
                sem_g,
            )

    def drain_gathers(s):
        for off, ln in _SEGS:
            pltpu.make_async_copy(
                table_h.at[wl_v.at[pl.ds(s * NWIN + off, ln)]],
                win_v.at[pl.ds(s * NWIN + off, ln)],
                sem_g,
            ).wait()

    def out_descs(g, s):
        bbase = wid * BPW + g * BB
        return [
            pltpu.make_async_copy(
                out_v.at[pl.ds(s * BLK + k * L, L)],
                out_h.at[bbase + k],
                sem_o,
            )
            for k in range(BB)
        ]

    stage(0, 0)
    fire_gathers(0)

    def block_body(g, carry):
        s = lax.rem(g, 2)
        s1 = 1 - s

        @pl.when(g + 1 < NBLK)
        def _():
            stage(g + 1, s1)

            @pl.when(g >= 1)
            def _():
                for d in out_descs(g - 1, s1):
                    d.wait()

            fire_gathers(s1)

        drain_gathers(s)

        @plsc.parallel_loop(0, BLK, unroll=8)
        def _(r):
            r_b = jnp.full((LANES,), r, jnp.int32)
            o_b = plsc.load_gather(o_v, [s * BLK + r_b])
            dur = plsc.load_gather(dur_v, [s * BLK + r_b])
            p = o_b + c_emb
            hi = (p >= EMB).astype(jnp.int32)
            seg = (2 * r_b + hi) >> 7
            phys = (s * NWIN * HID + 28 * r) + p + (seg << 8)
            emb = plsc.load_gather(win_v, [phys >> 4, phys & 15])
            plsc.store_scatter(
                out_v, [s * BLK + r_b, lane], jnp.where(m_emb, emb, dur)
            )

        for d in out_descs(g, s):
            d.start()
        return carry

    lax.fori_loop(0, NBLK, block_body, 0)

    for d in out_descs(NBLK - 2, lax.rem(NBLK - 2, 2)):
        d.wait()
    for d in out_descs(NBLK - 1, lax.rem(NBLK - 1, 2)):
        d.wait()


def kernel(x, table):
    idx = x[..., 0].astype(jnp.int32).reshape(N)
    dur = x[..., 1].reshape(N)
    return _sc_embed(table, idx, dur)


# speedup vs baseline: 1.2451x; 1.2451x over previous
import functools

import jax
import jax.numpy as jnp
from jax import lax
from jax.experimental import pallas as pl
from jax.experimental.pallas import tpu as pltpu
from jax.experimental.pallas import tpu_sc as plsc

B, L = 4096, 200
EMB = 14
HID = 16
N = B * L
VOC = 1000000
VROW = VOC * EMB // HID
W_CLAMP = VROW - 1

_info = plsc.get_sparse_core_info()
NC, NS, LANES = _info.num_cores, _info.num_subcores, _info.num_lanes
NW = NC * NS
BPW = B // NW
BB = 4
BLK = BB * L
NBLK = BPW // BB
PER_W = BPW * L
NWIN = 2 * BLK
_SEGS = [(s, min(128, NWIN - s)) for s in range(0, NWIN, 128)]

_mesh = plsc.VectorSubcoreMesh(core_axis_name="c", subcore_axis_name="s")


@functools.partial(
    pl.kernel,
    mesh=_mesh,
    out_type=jax.ShapeDtypeStruct((B, L, HID), jnp.float32),
    scratch_types=[
        pltpu.VMEM((2 * BLK,), jnp.int32),
        pltpu.VMEM((2 * BLK,), jnp.int32),
        pltpu.VMEM((2 * BLK,), jnp.int32),
        pltpu.VMEM((2 * BLK,), jnp.float32),
        pltpu.VMEM((2 * NWIN,), jnp.int32),
        pltpu.VMEM((2 * NWIN, HID), jnp.float32),
        pltpu.VMEM((2 * BLK, HID), jnp.float32),
        pltpu.SemaphoreType.DMA,
        pltpu.SemaphoreType.DMA,
    ],
    compiler_params=pltpu.CompilerParams(
        needs_layout_passes=False,
        use_tc_tiling_on_sc=False,
    ),
)
def _sc_embed(table_h, idx_h, dur_h, out_h, idx_v, w1_v, o_v, dur_v, wl_v,
              win_v, out_v, sem_g, sem_o):
    wid = lax.axis_index("s") * NC + lax.axis_index("c")
    lane = lax.iota(jnp.int32, LANES)
    c_emb = jnp.minimum(lane, EMB - 1)
    m_emb = lane < EMB

    def stage(g, s):
        base = wid * PER_W + g * BLK
        pltpu.sync_copy(idx_h.at[pl.ds(base, BLK)], idx_v.at[pl.ds(s * BLK, BLK)])
        pltpu.sync_copy(dur_h.at[pl.ds(base, BLK)], dur_v.at[pl.ds(s * BLK, BLK)])

        @plsc.parallel_loop(0, BLK // LANES, unroll=4)
        def _(j):
            ii = idx_v[pl.ds(s * BLK + j * LANES, LANES)]
            w = ii * EMB
            w1_v[pl.ds(s * BLK + j * LANES, LANES)] = w >> 4
            o_v[pl.ds(s * BLK + j * LANES, LANES)] = w & 15

        @plsc.parallel_loop(0, NWIN // LANES, unroll=4)
        def _(j):
            k = j * LANES + lane
            t = plsc.load_gather(w1_v, [s * BLK + (k >> 1)])
            wl_v[pl.ds(s * NWIN + j * LANES, LANES)] = jnp.minimum(
                t + (k & 1), W_CLAMP
            )

    def fire_gathers(s):
        for off, ln in _SEGS:
            pltpu.async_copy(
                table_h.at[wl_v.at[pl.ds(s * NWIN + off, ln)]],
                win_v.at[pl.ds(s * NWIN + off, ln)],
                sem_g,
            )

    def drain_gathers(s):
        for off, ln in _SEGS:
            pltpu.make_async_copy(
                table_h.at[wl_v.at[pl.ds(s * NWIN + off, ln)]],
                win_v.at[pl.ds(s * NWIN + off, ln)],
                sem_g,
            ).wait()

    def out_descs(g, s):
        bbase = wid * BPW + g * BB
        return [
            pltpu.make_async_copy(
                out_v.at[pl.ds(s * BLK + k * L, L)],
                out_h.at[bbase + k],
                sem_o,
            )
            for k in range(BB)
        ]

    stage(0, 0)
    fire_gathers(0)

    def block_body(g, carry):
        s = lax.rem(g, 2)
        s1 = 1 - s

        @pl.when(g + 1 < NBLK)
        def _():
            stage(g + 1, s1)

            @pl.when(g >= 1)
            def _():
                for d in out_descs(g - 1, s1):
                    d.wait()

            fire_gathers(s1)

        drain_gathers(s)

        @plsc.parallel_loop(0, BLK, unroll=8)
        def _(r):
            r_b = jnp.full((LANES,), r, jnp.int32)
            o_b = plsc.load_gather(o_v, [s * BLK + r_b])
            dur = plsc.load_gather(dur_v, [s * BLK + r_b])
            p = o_b + c_emb
            emb = plsc.load_gather(win_v, [s * NWIN + 2 * r_b + (p >> 4), p & 15])
            plsc.store_scatter(
                out_v, [s * BLK + r_b, lane], jnp.where(m_emb, emb, dur)
            )

        for d in out_descs(g, s):
            d.start()
        return carry

    lax.fori_loop(0, NBLK, block_body, 0)

    for d in out_descs(NBLK - 2, lax.rem(NBLK - 2, 2)):
        d.wait()
    for d in out_descs(NBLK - 1, lax.rem(NBLK - 1, 2)):
        d.wait()


def kernel(x, table):
    table16 = table.reshape(VROW, HID)
    idx = x[..., 0].astype(jnp.int32).reshape(N)
    dur = x[..., 1].reshape(N)
    return _sc_embed(table16, idx, dur)


# speedup vs baseline: 1.2459x; 1.0006x over previous
import functools

import jax
import jax.numpy as jnp
from jax import lax
from jax.experimental import pallas as pl
from jax.experimental.pallas import tpu as pltpu
from jax.experimental.pallas import tpu_sc as plsc

B, L = 4096, 200
EMB = 14
HID = 16
N = B * L
VOC = 1000000
VROW = VOC * EMB // HID
W_CLAMP = VROW - 1

_info = plsc.get_sparse_core_info()
NC, NS, LANES = _info.num_cores, _info.num_subcores, _info.num_lanes
NW = NC * NS
BPW = B // NW
BB = 4
BLK = BB * L
NBLK = BPW // BB
PER_W = BPW * L
NWIN = 2 * BLK
_SEGS = [(s, min(128, NWIN - s)) for s in range(0, NWIN, 128)]

_mesh = plsc.VectorSubcoreMesh(core_axis_name="c", subcore_axis_name="s")


@functools.partial(
    pl.kernel,
    mesh=_mesh,
    out_type=jax.ShapeDtypeStruct((B, L, HID), jnp.float32),
    scratch_types=[
        pltpu.VMEM((2 * BLK,), jnp.int32),
        pltpu.VMEM((2 * BLK,), jnp.int32),
        pltpu.VMEM((2 * BLK,), jnp.int32),
        pltpu.VMEM((2 * BLK,), jnp.float32),
        pltpu.VMEM((2 * NWIN,), jnp.int32),
        pltpu.VMEM((2 * NWIN, HID), jnp.float32),
        pltpu.VMEM((2 * BLK, HID), jnp.float32),
        pltpu.SemaphoreType.DMA,
        pltpu.SemaphoreType.DMA,
    ],
    compiler_params=pltpu.CompilerParams(
        needs_layout_passes=False,
        use_tc_tiling_on_sc=False,
    ),
)
def _sc_embed(table_h, idx_h, dur_h, out_h, idx_v, w1_v, o_v, dur_v, wl_v,
              win_v, out_v, sem_g, sem_o):
    wid = lax.axis_index("s") * NC + lax.axis_index("c")
    lane = lax.iota(jnp.int32, LANES)
    c_emb = jnp.minimum(lane, EMB - 1)
    m_emb = lane < EMB

    def stage(g, s):
        base = wid * PER_W + g * BLK
        pltpu.sync_copy(idx_h.at[pl.ds(base, BLK)], idx_v.at[pl.ds(s * BLK, BLK)])
        pltpu.sync_copy(dur_h.at[pl.ds(base, BLK)], dur_v.at[pl.ds(s * BLK, BLK)])

        @plsc.parallel_loop(0, BLK // LANES, unroll=4)
        def _(j):
            ii = idx_v[pl.ds(s * BLK + j * LANES, LANES)]
            w = ii * EMB
            w1_v[pl.ds(s * BLK + j * LANES, LANES)] = w >> 4
            o_v[pl.ds(s * BLK + j * LANES, LANES)] = w & 15

        @plsc.parallel_loop(0, NWIN // LANES, unroll=4)
        def _(j):
            k = j * LANES + lane
            t = plsc.load_gather(w1_v, [s * BLK + (k >> 1)])
            wl_v[pl.ds(s * NWIN + j * LANES, LANES)] = jnp.minimum(
                t + (k & 1), W_CLAMP
            )

    def fire_gathers(s):
        for off, ln in _SEGS:
            pltpu.async_copy(
                table_h.at[wl_v.at[pl.ds(s * NWIN + off, ln)]],
                win_v.at[pl.ds(s * NWIN + off, ln)],
                sem_g,
            )

    def drain_gathers(s):
        for off, ln in _SEGS:
            pltpu.make_async_copy(
                table_h.at[wl_v.at[pl.ds(s * NWIN + off, ln)]],
                win_v.at[pl.ds(s * NWIN + off, ln)],
                sem_g,
            ).wait()

    def out_descs(g, s):
        bbase = wid * BPW + g * BB
        return [
            pltpu.make_async_copy(
                out_v.at[pl.ds(s * BLK + k * L, L)],
                out_h.at[bbase + k],
                sem_o,
            )
            for k in range(BB)
        ]

    stage(0, 0)
    fire_gathers(0)

    def block_body(g, carry):
        s = lax.rem(g, 2)
        s1 = 1 - s

        @pl.when(g + 1 < NBLK)
        def _():
            stage(g + 1, s1)

            @pl.when(g >= 1)
            def _():
                for d in out_descs(g - 1, s1):
                    d.wait()

            fire_gathers(s1)

        drain_gathers(s)

        @plsc.parallel_loop(0, BLK, unroll=16)
        def _(r):
            r_b = jnp.full((LANES,), r, jnp.int32)
            o_b = plsc.load_gather(o_v, [s * BLK + r_b])
            dur = plsc.load_gather(dur_v, [s * BLK + r_b])
            p = o_b + c_emb
            emb = plsc.load_gather(win_v, [s * NWIN + 2 * r_b + (p >> 4), p & 15])
            plsc.store_scatter(
                out_v, [s * BLK + r_b, lane], jnp.where(m_emb, emb, dur)
            )

        for d in out_descs(g, s):
            d.start()
        return carry

    lax.fori_loop(0, NBLK, block_body, 0)

    for d in out_descs(NBLK - 2, lax.rem(NBLK - 2, 2)):
        d.wait()
    for d in out_descs(NBLK - 1, lax.rem(NBLK - 1, 2)):
        d.wait()


def kernel(x, table):
    table16 = table.reshape(VROW, HID)
    idx = x[..., 0].astype(jnp.int32).reshape(N)
    dur = x[..., 1].reshape(N)
    return _sc_embed(table16, idx, dur)
